# Initial kernel scaffold; baseline (speedup 1.0000x reference)
import jax
import jax.numpy as jnp
from jax.experimental import pallas as pl


def kernel(in_feat, edge_index, d, h0, W, lamda, alpha, l):
    raise NotImplementedError("write your pallas kernel here")


# speedup vs baseline: 5.0332x; 5.0332x over previous
import functools

import jax
import jax.numpy as jnp
from jax import lax
from jax.experimental import pallas as pl
from jax.experimental.pallas import tpu as pltpu
from jax.experimental.pallas import tpu_sc as plsc

_N = 10000
_NP = 10240
_E = 320000
_D = 128
_NC = 2
_NS = 16
_ROWS_PER_TILE = _NP // _NS
_EDGES_PER_TILE = _E // (_NC * _NS)
_K = 80
_NCHUNK = _EDGES_PER_TILE // _K
_ZR = 128


def _scale_body(x_ref, d_ref, o_ref):
    o_ref[...] = x_ref[...] * d_ref[...]


def _combine_body(s_ref, p0_ref, p1_ref, d_ref, h0_ref, w_ref, o_ref):
    theta = s_ref[0]
    alpha = s_ref[1]
    agg = p0_ref[...] + p1_ref[...]
    support = (1.0 - alpha) * (agg * d_ref[...]) + alpha * h0_ref[...]
    o_ref[...] = theta * jnp.dot(
        support, w_ref[...], preferred_element_type=jnp.float32
    ) + (1.0 - theta) * support


def _sc_segment_sum_body(h_hbm, src_hbm, dst_hbm, out_hbm,
                         idx_s, idx_d, rows, zbuf, agg_sh, sem):
    c = lax.axis_index("c")
    s = lax.axis_index("s")

    def _zero_row(i, carry):
        for j in range(_D // 16):
            zbuf[i, pl.ds(j * 16, 16)] = jnp.zeros((16,), jnp.float32)
        return carry

    lax.fori_loop(0, _ZR, _zero_row, 0)
    row0 = s * _ROWS_PER_TILE
    for i in range(_ROWS_PER_TILE // _ZR):
        pltpu.sync_copy(zbuf, agg_sh.at[pl.ds(row0 + i * _ZR, _ZR), :])
    plsc.subcore_barrier()

    base_e = (c * _NS + s) * _EDGES_PER_TILE

    def _chunk(j, carry):
        off = pl.multiple_of(base_e + j * _K, 8)
        pltpu.sync_copy(src_hbm.at[pl.ds(off, _K)], idx_s)
        pltpu.sync_copy(dst_hbm.at[pl.ds(off, _K)], idx_d)
        pltpu.async_copy(h_hbm.at[idx_s], rows, sem).wait()
        pltpu.sync_copy(rows, agg_sh.at[idx_d], add=True)
        return carry

    lax.fori_loop(0, _NCHUNK, _chunk, 0)
    plsc.subcore_barrier()

    o0 = pl.multiple_of(c * _NP + row0, 8)
    pltpu.sync_copy(agg_sh.at[pl.ds(row0, _ROWS_PER_TILE), :],
                    out_hbm.at[pl.ds(o0, _ROWS_PER_TILE), :])


_sc_segment_sum = functools.partial(
    pl.kernel,
    out_type=jax.ShapeDtypeStruct((_NC * _NP, _D), jnp.float32),
    mesh=plsc.VectorSubcoreMesh(
        core_axis_name="c", subcore_axis_name="s",
        num_cores=_NC, num_subcores=_NS),
    scratch_types=[
        pltpu.VMEM((_K,), jnp.int32),
        pltpu.VMEM((_K,), jnp.int32),
        pltpu.VMEM((_K, _D), jnp.float32),
        pltpu.VMEM((_ZR, _D), jnp.float32),
        pltpu.VMEM_SHARED((_NP, _D), jnp.float32),
        pltpu.SemaphoreType.DMA,
    ],
)(_sc_segment_sum_body)


def kernel(in_feat, edge_index, d, h0, W, lamda, alpha, l):
    src = edge_index[0].astype(jnp.int32)
    dst = edge_index[1].astype(jnp.int32)
    d2 = d[:, None]
    theta = jnp.log(lamda / l + 1.0)
    scals = jnp.stack([theta, alpha]).astype(jnp.float32)

    bn = 400
    grid = _N // bn

    h = pl.pallas_call(
        _scale_body,
        grid=(grid,),
        in_specs=[
            pl.BlockSpec((bn, _D), lambda i: (i, 0)),
            pl.BlockSpec((bn, 1), lambda i: (i, 0)),
        ],
        out_specs=pl.BlockSpec((bn, _D), lambda i: (i, 0)),
        out_shape=jax.ShapeDtypeStruct((_N, _D), jnp.float32),
    )(in_feat, d2)

    partials = _sc_segment_sum(h, src, dst)
    p0 = partials[:_N]
    p1 = partials[_NP:_NP + _N]

    out = pl.pallas_call(
        _combine_body,
        grid=(grid,),
        in_specs=[
            pl.BlockSpec(memory_space=pltpu.SMEM),
            pl.BlockSpec((bn, _D), lambda i: (i, 0)),
            pl.BlockSpec((bn, _D), lambda i: (i, 0)),
            pl.BlockSpec((bn, 1), lambda i: (i, 0)),
            pl.BlockSpec((bn, _D), lambda i: (i, 0)),
            pl.BlockSpec((_D, _D), lambda i: (0, 0)),
        ],
        out_specs=pl.BlockSpec((bn, _D), lambda i: (i, 0)),
        out_shape=jax.ShapeDtypeStruct((_N, _D), jnp.float32),
    )(scals, p0, p1, d2, h0, W)
    return out


# speedup vs baseline: 10.5442x; 2.0949x over previous
import functools

import jax
import jax.numpy as jnp
from jax import lax
from jax.experimental import pallas as pl
from jax.experimental.pallas import tpu as pltpu
from jax.experimental.pallas import tpu_sc as plsc

_N = 10000
_NP = 10112
_E = 320000
_D = 128
_NC = 2
_NS = 16
_NT = _NC * _NS
_ROWS_PER_TILE = _NP // _NS
_KP = 128
_NCH = 79
_EPT = _KP * _NCH
_EP = _NT * _EPT
_NBUF = 3
_STEPS = _NCH + 3
_NGRP = (_STEPS + _NBUF - 1) // _NBUF


def _scale_body(x_ref, d_ref, o_ref):
    o_ref[...] = x_ref[...] * d_ref[...]


def _combine_body(s_ref, p0_ref, p1_ref, d_ref, h0_ref, w_ref, o_ref):
    theta = s_ref[0]
    alpha = s_ref[1]
    agg = p0_ref[...] + p1_ref[...]
    support = (1.0 - alpha) * (agg * d_ref[...]) + alpha * h0_ref[...]
    o_ref[...] = theta * jnp.dot(
        support, w_ref[...], preferred_element_type=jnp.float32
    ) + (1.0 - theta) * support


def _sc_segment_sum_body(h_hbm, src_hbm, dst_hbm, out_hbm,
                         is0, is1, is2, id0, id1, id2,
                         r0, r1, r2, agg_sh,
                         q0, q1, q2, g0, g1, g2):
    c = lax.axis_index("c")
    s = lax.axis_index("s")
    wid = c * _NS + s
    ibs = (is0, is1, is2)
    ibd = (id0, id1, id2)
    rows = (r0, r1, r2)
    isem = (q0, q1, q2)
    gsem = (g0, g1, g2)

    def _zero_row(i, carry):
        for j in range(_D // 16):
            r0[i, pl.ds(j * 16, 16)] = jnp.zeros((16,), jnp.float32)
        return carry

    lax.fori_loop(0, _KP, _zero_row, 0)
    row0 = s * _ROWS_PER_TILE
    nfull = _ROWS_PER_TILE // _KP
    for i in range(nfull):
        pltpu.sync_copy(r0, agg_sh.at[pl.ds(row0 + i * _KP, _KP), :])
    rem = _ROWS_PER_TILE - nfull * _KP
    if rem:
        pltpu.sync_copy(r0.at[pl.ds(0, rem), :],
                        agg_sh.at[pl.ds(row0 + nfull * _KP, rem), :])
    plsc.subcore_barrier()

    cbase = wid * _NCH

    def _group(g, carry):
        for b in range(_NBUF):
            step = g * _NBUF + b
            bp = (b + 2) % _NBUF

            @pl.when(jnp.logical_and(step >= 3, step < _NCH + 3))
            def _():
                t = step - 3
                pltpu.make_async_copy(
                    h_hbm.at[ibs[b]], rows[b], gsem[b]).wait()
                pltpu.sync_copy(rows[b], agg_sh.at[ibd[b]], add=True)

            @pl.when(step < _NCH)
            def _():
                ci = cbase + step
                pltpu.async_copy(src_hbm.at[ci], ibs[b], isem[b])
                pltpu.async_copy(dst_hbm.at[ci], ibd[b], isem[b])

            @pl.when(jnp.logical_and(step >= 1, step < _NCH + 1))
            def _():
                cp = cbase + step - 1
                pltpu.make_async_copy(src_hbm.at[cp], ibs[bp], isem[bp]).wait()
                pltpu.make_async_copy(dst_hbm.at[cp], ibd[bp], isem[bp]).wait()
                pltpu.async_copy(h_hbm.at[ibs[bp]], rows[bp], gsem[bp])
        return carry

    lax.fori_loop(0, _NGRP, _group, 0)
    plsc.subcore_barrier()

    o0 = pl.multiple_of(c * _NP + row0, 8)
    pltpu.sync_copy(agg_sh.at[pl.ds(row0, _ROWS_PER_TILE), :],
                    out_hbm.at[pl.ds(o0, _ROWS_PER_TILE), :])


_sc_segment_sum = functools.partial(
    pl.kernel,
    out_type=jax.ShapeDtypeStruct((_NC * _NP, _D), jnp.float32),
    mesh=plsc.VectorSubcoreMesh(
        core_axis_name="c", subcore_axis_name="s",
        num_cores=_NC, num_subcores=_NS),
    scratch_types=[
        pltpu.VMEM((_KP,), jnp.int32),
        pltpu.VMEM((_KP,), jnp.int32),
        pltpu.VMEM((_KP,), jnp.int32),
        pltpu.VMEM((_KP,), jnp.int32),
        pltpu.VMEM((_KP,), jnp.int32),
        pltpu.VMEM((_KP,), jnp.int32),
        pltpu.VMEM((_KP, _D), jnp.float32),
        pltpu.VMEM((_KP, _D), jnp.float32),
        pltpu.VMEM((_KP, _D), jnp.float32),
        pltpu.VMEM_SHARED((_NP, _D), jnp.float32),
        pltpu.SemaphoreType.DMA,
        pltpu.SemaphoreType.DMA,
        pltpu.SemaphoreType.DMA,
        pltpu.SemaphoreType.DMA,
        pltpu.SemaphoreType.DMA,
        pltpu.SemaphoreType.DMA,
    ],
)(_sc_segment_sum_body)


def kernel(in_feat, edge_index, d, h0, W, lamda, alpha, l):
    src = edge_index[0].astype(jnp.int32)
    dst = edge_index[1].astype(jnp.int32)
    d2 = d[:, None]
    theta = jnp.log(lamda / l + 1.0)
    scals = jnp.stack([theta, alpha]).astype(jnp.float32)

    pad = _N + (jnp.arange(_EP - _E, dtype=jnp.int32) % (_NP - _N))
    src_p = jnp.concatenate([src, pad]).reshape(_NT * _NCH, _KP)
    dst_p = jnp.concatenate([dst, pad]).reshape(_NT * _NCH, _KP)

    bn = 400
    grid = _N // bn

    h = pl.pallas_call(
        _scale_body,
        grid=(grid,),
        in_specs=[
            pl.BlockSpec((bn, _D), lambda i: (i, 0)),
            pl.BlockSpec((bn, 1), lambda i: (i, 0)),
        ],
        out_specs=pl.BlockSpec((bn, _D), lambda i: (i, 0)),
        out_shape=jax.ShapeDtypeStruct((_NP, _D), jnp.float32),
    )(in_feat, d2)

    partials = _sc_segment_sum(h, src_p, dst_p)
    p0 = partials[:_N]
    p1 = partials[_NP:_NP + _N]

    out = pl.pallas_call(
        _combine_body,
        grid=(grid,),
        in_specs=[
            pl.BlockSpec(memory_space=pltpu.SMEM),
            pl.BlockSpec((bn, _D), lambda i: (i, 0)),
            pl.BlockSpec((bn, _D), lambda i: (i, 0)),
            pl.BlockSpec((bn, 1), lambda i: (i, 0)),
            pl.BlockSpec((bn, _D), lambda i: (i, 0)),
            pl.BlockSpec((_D, _D), lambda i: (0, 0)),
        ],
        out_specs=pl.BlockSpec((bn, _D), lambda i: (i, 0)),
        out_shape=jax.ShapeDtypeStruct((_N, _D), jnp.float32),
    )(scals, p0, p1, d2, h0, W)
    return out


# speedup vs baseline: 11.0634x; 1.0492x over previous
import functools

import jax
import jax.numpy as jnp
from jax import lax
from jax.experimental import pallas as pl
from jax.experimental.pallas import tpu as pltpu
from jax.experimental.pallas import tpu_sc as plsc

_N = 10000
_NP = 10112
_E = 320000
_D = 128
_NC = 2
_NS = 16
_NT = _NC * _NS
_ROWS_PER_TILE = _NP // _NS
_EPT = _E // _NT
_KP = 128
_NCH = _EPT // _KP
_KT = _EPT - _NCH * _KP
_NBUF = 3
_STEPS = _NCH + 3
_NGRP = (_STEPS + _NBUF - 1) // _NBUF


def _scale_body(x_ref, d_ref, o_ref):
    o_ref[...] = x_ref[...] * d_ref[...]


def _combine_body(s_ref, p0_ref, p1_ref, d_ref, h0_ref, w_ref, o_ref):
    theta = s_ref[0]
    alpha = s_ref[1]
    agg = p0_ref[0] + p1_ref[0]
    support = (1.0 - alpha) * (agg * d_ref[...]) + alpha * h0_ref[...]
    o_ref[...] = theta * jnp.dot(
        support, w_ref[...], preferred_element_type=jnp.float32
    ) + (1.0 - theta) * support


def _sc_segment_sum_body(h_hbm, src_hbm, dst_hbm, out_hbm,
                         is0, is1, is2, id0, id1, id2, ist, idt,
                         r0, r1, r2, agg_sh,
                         q0, q1, q2, g0, g1, g2):
    c = lax.axis_index("c")
    s = lax.axis_index("s")
    wid = c * _NS + s
    ibs = (is0, is1, is2)
    ibd = (id0, id1, id2)
    rows = (r0, r1, r2)
    isem = (q0, q1, q2)
    gsem = (g0, g1, g2)

    def _zero_row(i, carry):
        for j in range(_D // 16):
            r0[i, pl.ds(j * 16, 16)] = jnp.zeros((16,), jnp.float32)
        return carry

    lax.fori_loop(0, _KP, _zero_row, 0)
    row0 = s * _ROWS_PER_TILE
    nfull = _ROWS_PER_TILE // _KP
    for i in range(nfull):
        pltpu.sync_copy(r0, agg_sh.at[pl.ds(row0 + i * _KP, _KP), :])
    rem = _ROWS_PER_TILE - nfull * _KP
    if rem:
        pltpu.sync_copy(r0.at[pl.ds(0, rem), :],
                        agg_sh.at[pl.ds(row0 + nfull * _KP, rem), :])
    plsc.subcore_barrier()

    ebase = wid * _EPT

    def _group(g, carry):
        for b in range(_NBUF):
            step = g * _NBUF + b
            bp = (b + 2) % _NBUF

            @pl.when(jnp.logical_and(step >= 3, step < _NCH + 3))
            def _():
                pltpu.make_async_copy(
                    h_hbm.at[ibs[b]], rows[b], gsem[b]).wait()
                pltpu.sync_copy(rows[b], agg_sh.at[ibd[b]], add=True)

            @pl.when(step < _NCH)
            def _():
                off = pl.multiple_of(ebase + step * _KP, 8)
                pltpu.async_copy(src_hbm.at[pl.ds(off, _KP)], ibs[b], isem[b])
                pltpu.async_copy(dst_hbm.at[pl.ds(off, _KP)], ibd[b], isem[b])

            @pl.when(jnp.logical_and(step >= 1, step < _NCH + 1))
            def _():
                off = pl.multiple_of(ebase + (step - 1) * _KP, 8)
                pltpu.make_async_copy(
                    src_hbm.at[pl.ds(off, _KP)], ibs[bp], isem[bp]).wait()
                pltpu.make_async_copy(
                    dst_hbm.at[pl.ds(off, _KP)], ibd[bp], isem[bp]).wait()
                pltpu.async_copy(h_hbm.at[ibs[bp]], rows[bp], gsem[bp])
        return carry

    lax.fori_loop(0, _NGRP, _group, 0)

    toff = pl.multiple_of(ebase + _NCH * _KP, 8)
    pltpu.sync_copy(src_hbm.at[pl.ds(toff, _KT)], ist)
    pltpu.sync_copy(dst_hbm.at[pl.ds(toff, _KT)], idt)
    pltpu.async_copy(h_hbm.at[ist], r0.at[pl.ds(0, _KT), :], g0).wait()
    pltpu.sync_copy(r0.at[pl.ds(0, _KT), :], agg_sh.at[idt], add=True)
    plsc.subcore_barrier()

    o0 = pl.multiple_of(c * _NP + row0, 8)
    pltpu.sync_copy(agg_sh.at[pl.ds(row0, _ROWS_PER_TILE), :],
                    out_hbm.at[pl.ds(o0, _ROWS_PER_TILE), :])


_sc_segment_sum = functools.partial(
    pl.kernel,
    out_type=jax.ShapeDtypeStruct((_NC * _NP, _D), jnp.float32),
    mesh=plsc.VectorSubcoreMesh(
        core_axis_name="c", subcore_axis_name="s",
        num_cores=_NC, num_subcores=_NS),
    scratch_types=[
        pltpu.VMEM((_KP,), jnp.int32),
        pltpu.VMEM((_KP,), jnp.int32),
        pltpu.VMEM((_KP,), jnp.int32),
        pltpu.VMEM((_KP,), jnp.int32),
        pltpu.VMEM((_KP,), jnp.int32),
        pltpu.VMEM((_KP,), jnp.int32),
        pltpu.VMEM((_KT,), jnp.int32),
        pltpu.VMEM((_KT,), jnp.int32),
        pltpu.VMEM((_KP, _D), jnp.float32),
        pltpu.VMEM((_KP, _D), jnp.float32),
        pltpu.VMEM((_KP, _D), jnp.float32),
        pltpu.VMEM_SHARED((_NP, _D), jnp.float32),
        pltpu.SemaphoreType.DMA,
        pltpu.SemaphoreType.DMA,
        pltpu.SemaphoreType.DMA,
        pltpu.SemaphoreType.DMA,
        pltpu.SemaphoreType.DMA,
        pltpu.SemaphoreType.DMA,
    ],
)(_sc_segment_sum_body)


def kernel(in_feat, edge_index, d, h0, W, lamda, alpha, l):
    src = edge_index[0].astype(jnp.int32)
    dst = edge_index[1].astype(jnp.int32)
    d2 = d[:, None]
    theta = jnp.log(lamda / l + 1.0)
    scals = jnp.stack([theta, alpha]).astype(jnp.float32)

    bn = 400
    grid = _N // bn

    h = pl.pallas_call(
        _scale_body,
        grid=(grid,),
        in_specs=[
            pl.BlockSpec((bn, _D), lambda i: (i, 0)),
            pl.BlockSpec((bn, 1), lambda i: (i, 0)),
        ],
        out_specs=pl.BlockSpec((bn, _D), lambda i: (i, 0)),
        out_shape=jax.ShapeDtypeStruct((_N, _D), jnp.float32),
    )(in_feat, d2)

    partials = _sc_segment_sum(h, src, dst).reshape(_NC, _NP, _D)

    out = pl.pallas_call(
        _combine_body,
        grid=(grid,),
        in_specs=[
            pl.BlockSpec(memory_space=pltpu.SMEM),
            pl.BlockSpec((1, bn, _D), lambda i: (0, i, 0)),
            pl.BlockSpec((1, bn, _D), lambda i: (1, i, 0)),
            pl.BlockSpec((bn, 1), lambda i: (i, 0)),
            pl.BlockSpec((bn, _D), lambda i: (i, 0)),
            pl.BlockSpec((_D, _D), lambda i: (0, 0)),
        ],
        out_specs=pl.BlockSpec((bn, _D), lambda i: (i, 0)),
        out_shape=jax.ShapeDtypeStruct((_N, _D), jnp.float32),
    )(scals, partials, partials, d2, h0, W)
    return out


# speedup vs baseline: 12.6171x; 1.1404x over previous
import functools

import jax
import jax.numpy as jnp
from jax import lax
from jax.experimental import pallas as pl
from jax.experimental.pallas import tpu as pltpu
from jax.experimental.pallas import tpu_sc as plsc

_N = 10000
_NP = 10112
_E = 320000
_D = 128
_NC = 2
_NS = 16
_NT = _NC * _NS
_ROWS_PER_TILE = _NP // _NS
_EPT = _E // _NT
_KP = 128
_NCH = _EPT // _KP
_KT = _EPT - _NCH * _KP
_NBUF = 3
_STEPS = _NCH + 3
_NGRP = (_STEPS + _NBUF - 1) // _NBUF


def _scale_body(x_ref, d_ref, o_ref):
    dcol = jnp.transpose(d_ref[...], (1, 0))
    o_ref[...] = x_ref[...] * dcol


def _combine_body(s_ref, p0_ref, p1_ref, d_ref, h0_ref, w_ref, o_ref):
    theta = s_ref[0]
    alpha = s_ref[1]
    agg = p0_ref[0] + p1_ref[0]
    dcol = jnp.transpose(d_ref[...], (1, 0))
    support = (1.0 - alpha) * (agg * dcol) + alpha * h0_ref[...]
    o_ref[...] = theta * jnp.dot(
        support, w_ref[...], preferred_element_type=jnp.float32
    ) + (1.0 - theta) * support


def _sc_segment_sum_body(h_hbm, src_hbm, dst_hbm, out_hbm,
                         is0, is1, is2, id0, id1, id2, ist, idt,
                         r0, r1, r2, agg_sh,
                         q0, q1, q2, g0, g1, g2):
    c = lax.axis_index("c")
    s = lax.axis_index("s")
    wid = c * _NS + s
    ibs = (is0, is1, is2)
    ibd = (id0, id1, id2)
    rows = (r0, r1, r2)
    isem = (q0, q1, q2)
    gsem = (g0, g1, g2)

    def _zero_row(i, carry):
        for j in range(_D // 16):
            r0[i, pl.ds(j * 16, 16)] = jnp.zeros((16,), jnp.float32)
        return carry

    lax.fori_loop(0, _KP, _zero_row, 0)
    row0 = s * _ROWS_PER_TILE
    nfull = _ROWS_PER_TILE // _KP
    for i in range(nfull):
        pltpu.sync_copy(r0, agg_sh.at[pl.ds(row0 + i * _KP, _KP), :])
    rem = _ROWS_PER_TILE - nfull * _KP
    if rem:
        pltpu.sync_copy(r0.at[pl.ds(0, rem), :],
                        agg_sh.at[pl.ds(row0 + nfull * _KP, rem), :])
    plsc.subcore_barrier()

    ebase = wid * _EPT

    def _group(g, carry):
        for b in range(_NBUF):
            step = g * _NBUF + b
            bp = (b + 2) % _NBUF

            @pl.when(jnp.logical_and(step >= 3, step < _NCH + 3))
            def _():
                pltpu.make_async_copy(
                    h_hbm.at[ibs[b]], rows[b], gsem[b]).wait()
                pltpu.sync_copy(rows[b], agg_sh.at[ibd[b]], add=True)

            @pl.when(step < _NCH)
            def _():
                off = pl.multiple_of(ebase + step * _KP, 8)
                pltpu.async_copy(src_hbm.at[pl.ds(off, _KP)], ibs[b], isem[b])
                pltpu.async_copy(dst_hbm.at[pl.ds(off, _KP)], ibd[b], isem[b])

            @pl.when(jnp.logical_and(step >= 1, step < _NCH + 1))
            def _():
                off = pl.multiple_of(ebase + (step - 1) * _KP, 8)
                pltpu.make_async_copy(
                    src_hbm.at[pl.ds(off, _KP)], ibs[bp], isem[bp]).wait()
                pltpu.make_async_copy(
                    dst_hbm.at[pl.ds(off, _KP)], ibd[bp], isem[bp]).wait()
                pltpu.async_copy(h_hbm.at[ibs[bp]], rows[bp], gsem[bp])
        return carry

    lax.fori_loop(0, _NGRP, _group, 0)

    toff = pl.multiple_of(ebase + _NCH * _KP, 8)
    pltpu.sync_copy(src_hbm.at[pl.ds(toff, _KT)], ist)
    pltpu.sync_copy(dst_hbm.at[pl.ds(toff, _KT)], idt)
    pltpu.async_copy(h_hbm.at[ist], r0.at[pl.ds(0, _KT), :], g0).wait()
    pltpu.sync_copy(r0.at[pl.ds(0, _KT), :], agg_sh.at[idt], add=True)
    plsc.subcore_barrier()

    o0 = pl.multiple_of(c * _NP + row0, 8)
    pltpu.sync_copy(agg_sh.at[pl.ds(row0, _ROWS_PER_TILE), :],
                    out_hbm.at[pl.ds(o0, _ROWS_PER_TILE), :])


_sc_segment_sum = functools.partial(
    pl.kernel,
    out_type=jax.ShapeDtypeStruct((_NC * _NP, _D), jnp.float32),
    mesh=plsc.VectorSubcoreMesh(
        core_axis_name="c", subcore_axis_name="s",
        num_cores=_NC, num_subcores=_NS),
    scratch_types=[
        pltpu.VMEM((_KP,), jnp.int32),
        pltpu.VMEM((_KP,), jnp.int32),
        pltpu.VMEM((_KP,), jnp.int32),
        pltpu.VMEM((_KP,), jnp.int32),
        pltpu.VMEM((_KP,), jnp.int32),
        pltpu.VMEM((_KP,), jnp.int32),
        pltpu.VMEM((_KT,), jnp.int32),
        pltpu.VMEM((_KT,), jnp.int32),
        pltpu.VMEM((_KP, _D), jnp.float32),
        pltpu.VMEM((_KP, _D), jnp.float32),
        pltpu.VMEM((_KP, _D), jnp.float32),
        pltpu.VMEM_SHARED((_NP, _D), jnp.float32),
        pltpu.SemaphoreType.DMA,
        pltpu.SemaphoreType.DMA,
        pltpu.SemaphoreType.DMA,
        pltpu.SemaphoreType.DMA,
        pltpu.SemaphoreType.DMA,
        pltpu.SemaphoreType.DMA,
    ],
)(_sc_segment_sum_body)


def kernel(in_feat, edge_index, d, h0, W, lamda, alpha, l):
    src = edge_index[0].astype(jnp.int32)
    dst = edge_index[1].astype(jnp.int32)
    dr = d[None, :]
    theta = jnp.log(lamda / l + 1.0)
    scals = jnp.stack([theta, alpha]).astype(jnp.float32)

    h = pl.pallas_call(
        _scale_body,
        grid=(1,),
        in_specs=[
            pl.BlockSpec((_N, _D), lambda i: (0, 0)),
            pl.BlockSpec((1, _N), lambda i: (0, 0)),
        ],
        out_specs=pl.BlockSpec((_N, _D), lambda i: (0, 0)),
        out_shape=jax.ShapeDtypeStruct((_N, _D), jnp.float32),
    )(in_feat, dr)

    partials = _sc_segment_sum(h, src, dst).reshape(_NC, _NP, _D)

    out = pl.pallas_call(
        _combine_body,
        grid=(1,),
        in_specs=[
            pl.BlockSpec(memory_space=pltpu.SMEM),
            pl.BlockSpec((1, _N, _D), lambda i: (0, 0, 0)),
            pl.BlockSpec((1, _N, _D), lambda i: (1, 0, 0)),
            pl.BlockSpec((1, _N), lambda i: (0, 0)),
            pl.BlockSpec((_N, _D), lambda i: (0, 0)),
            pl.BlockSpec((_D, _D), lambda i: (0, 0)),
        ],
        out_specs=pl.BlockSpec((_N, _D), lambda i: (0, 0)),
        out_shape=jax.ShapeDtypeStruct((_N, _D), jnp.float32),
    )(scals, partials, partials, dr, h0, W)
    return out


# speedup vs baseline: 14.1745x; 1.1234x over previous
import functools

import jax
import jax.numpy as jnp
from jax import lax
from jax.experimental import pallas as pl
from jax.experimental.pallas import tpu as pltpu
from jax.experimental.pallas import tpu_sc as plsc

_N = 10000
_NP = 10112
_E = 320000
_D = 128
_NC = 2
_NS = 16
_NT = _NC * _NS
_ROWS_PER_TILE = _NP // _NS
_KP = 128
_NCHT = _E // _KP
_NCH0 = _NCHT // _NT
_NXT = _NCHT - _NCH0 * _NT
_NBUF = 3
_STEPS = _NCH0 + 1 + 3
_NGRP = (_STEPS + _NBUF - 1) // _NBUF


def _scale_body(x_ref, d_ref, o_ref):
    dcol = jnp.transpose(d_ref[...], (1, 0))
    o_ref[...] = x_ref[...] * dcol


def _combine_body(s_ref, p0_ref, p1_ref, d_ref, h0_ref, w_ref, o_ref):
    theta = s_ref[0]
    alpha = s_ref[1]
    agg = p0_ref[0] + p1_ref[0]
    dcol = jnp.transpose(d_ref[...], (1, 0))
    support = (1.0 - alpha) * (agg * dcol) + alpha * h0_ref[...]
    o_ref[...] = theta * jnp.dot(
        support, w_ref[...], preferred_element_type=jnp.float32
    ) + (1.0 - theta) * support


def _sc_segment_sum_body(h_hbm, ei_hbm, out_hbm,
                         is0, is1, is2,
                         r0, r1, r2, agg_sh,
                         q0, q1, q2, g0, g1, g2):
    c = lax.axis_index("c")
    s = lax.axis_index("s")
    wid = c * _NS + s
    ibs = (is0, is1, is2)
    rows = (r0, r1, r2)
    isem = (q0, q1, q2)
    gsem = (g0, g1, g2)

    def _zero_row(i, carry):
        for j in range(_D // 16):
            r0[i, pl.ds(j * 16, 16)] = jnp.zeros((16,), jnp.float32)
        return carry

    lax.fori_loop(0, _KP, _zero_row, 0)
    row0 = s * _ROWS_PER_TILE
    nfull = _ROWS_PER_TILE // _KP
    for i in range(nfull):
        pltpu.sync_copy(r0, agg_sh.at[pl.ds(row0 + i * _KP, _KP), :])
    rem = _ROWS_PER_TILE - nfull * _KP
    if rem:
        pltpu.sync_copy(r0.at[pl.ds(0, rem), :],
                        agg_sh.at[pl.ds(row0 + nfull * _KP, rem), :])
    plsc.subcore_barrier()

    nch = jnp.where(wid < _NXT, _NCH0 + 1, _NCH0)
    cb = wid * _NCH0 + jnp.minimum(wid, _NXT)

    def _group(g, carry):
        for b in range(_NBUF):
            step = g * _NBUF + b
            bp = (b + 2) % _NBUF

            @pl.when(jnp.logical_and(step >= 3, step < nch + 3))
            def _():
                pltpu.make_async_copy(
                    h_hbm.at[ibs[b].at[0]], rows[b], gsem[b]).wait()
                pltpu.sync_copy(rows[b], agg_sh.at[ibs[b].at[1]], add=True)

            @pl.when(step < nch)
            def _():
                off = pl.multiple_of((cb + step) * _KP, 128)
                pltpu.async_copy(ei_hbm.at[:, pl.ds(off, _KP)], ibs[b], isem[b])

            @pl.when(jnp.logical_and(step >= 1, step < nch + 1))
            def _():
                off = pl.multiple_of((cb + step - 1) * _KP, 128)
                pltpu.make_async_copy(
                    ei_hbm.at[:, pl.ds(off, _KP)], ibs[bp], isem[bp]).wait()
                pltpu.async_copy(h_hbm.at[ibs[bp].at[0]], rows[bp], gsem[bp])
        return carry

    lax.fori_loop(0, _NGRP, _group, 0)
    plsc.subcore_barrier()

    o0 = pl.multiple_of(c * _NP + row0, 8)
    pltpu.sync_copy(agg_sh.at[pl.ds(row0, _ROWS_PER_TILE), :],
                    out_hbm.at[pl.ds(o0, _ROWS_PER_TILE), :])


_sc_segment_sum = functools.partial(
    pl.kernel,
    out_type=jax.ShapeDtypeStruct((_NC * _NP, _D), jnp.float32),
    mesh=plsc.VectorSubcoreMesh(
        core_axis_name="c", subcore_axis_name="s",
        num_cores=_NC, num_subcores=_NS),
    scratch_types=[
        pltpu.VMEM((2, _KP), jnp.int32),
        pltpu.VMEM((2, _KP), jnp.int32),
        pltpu.VMEM((2, _KP), jnp.int32),
        pltpu.VMEM((_KP, _D), jnp.float32),
        pltpu.VMEM((_KP, _D), jnp.float32),
        pltpu.VMEM((_KP, _D), jnp.float32),
        pltpu.VMEM_SHARED((_NP, _D), jnp.float32),
        pltpu.SemaphoreType.DMA,
        pltpu.SemaphoreType.DMA,
        pltpu.SemaphoreType.DMA,
        pltpu.SemaphoreType.DMA,
        pltpu.SemaphoreType.DMA,
        pltpu.SemaphoreType.DMA,
    ],
)(_sc_segment_sum_body)


def kernel(in_feat, edge_index, d, h0, W, lamda, alpha, l):
    ei = edge_index.astype(jnp.int32)
    dr = d[None, :]
    theta = jnp.log(lamda / l + 1.0)
    scals = jnp.stack([theta, alpha]).astype(jnp.float32)

    h = pl.pallas_call(
        _scale_body,
        grid=(1,),
        in_specs=[
            pl.BlockSpec((_N, _D), lambda i: (0, 0)),
            pl.BlockSpec((1, _N), lambda i: (0, 0)),
        ],
        out_specs=pl.BlockSpec((_N, _D), lambda i: (0, 0)),
        out_shape=jax.ShapeDtypeStruct((_N, _D), jnp.float32),
    )(in_feat, dr)

    partials = _sc_segment_sum(h, ei).reshape(_NC, _NP, _D)

    out = pl.pallas_call(
        _combine_body,
        grid=(1,),
        in_specs=[
            pl.BlockSpec(memory_space=pltpu.SMEM),
            pl.BlockSpec((1, _N, _D), lambda i: (0, 0, 0)),
            pl.BlockSpec((1, _N, _D), lambda i: (1, 0, 0)),
            pl.BlockSpec((1, _N), lambda i: (0, 0)),
            pl.BlockSpec((_N, _D), lambda i: (0, 0)),
            pl.BlockSpec((_D, _D), lambda i: (0, 0)),
        ],
        out_specs=pl.BlockSpec((_N, _D), lambda i: (0, 0)),
        out_shape=jax.ShapeDtypeStruct((_N, _D), jnp.float32),
    )(scals, partials, partials, dr, h0, W)
    return out
